# baseline (speedup 1.0000x reference)
import functools

import jax
import jax.numpy as jnp
from jax import lax
from jax.experimental import pallas as pl
from jax.experimental.pallas import tpu as pltpu
from jax.experimental.pallas import tpu_sc as plsc

DIM = 768
EPS = 1e-9
L = 16
KV = DIM // L
NC, NS = 2, 16
NW = NC * NS
NSLOT = 4
G4 = 4


def _rsqrt(v):
    i = lax.bitcast_convert_type(v, jnp.int32)
    i = jnp.full((L,), 0x5F3759DF, jnp.int32) - lax.shift_right_logical(i, 1)
    y = lax.bitcast_convert_type(i, jnp.float32)
    half = v * 0.5
    for _ in range(3):
        y = y * (1.5 - half * y * y)
    return y


def _make_sc_kernel(T, N, C):
    RW = T // NW
    NCHUNK = RW // C
    mesh = plsc.VectorSubcoreMesh(core_axis_name="c", subcore_axis_name="s")

    @functools.partial(
        pl.kernel,
        out_type=jax.ShapeDtypeStruct((T, DIM), jnp.float32),
        mesh=mesh,
        compiler_params=pltpu.CompilerParams(needs_layout_passes=False),
        scratch_types=[
            pltpu.VMEM((RW,), jnp.int32),
            pltpu.VMEM((NSLOT, C, DIM), jnp.float32),
            pltpu.VMEM((NSLOT, C, DIM), jnp.float32),
            pltpu.VMEM((DIM,), jnp.float32),
            pltpu.VMEM((DIM,), jnp.float32),
            pltpu.VMEM((C, 2, L), jnp.float32),
            pltpu.SemaphoreType.DMA((NSLOT,)),
            pltpu.SemaphoreType.DMA((NSLOT,)),
            pltpu.SemaphoreType.DMA((NSLOT,)),
        ],
    )
    def body(ids_hbm, wt_hbm, pt_hbm, g_hbm, b_hbm, out_hbm,
             idx_v, wbuf, pbuf, gbuf, bbuf, red, sp, sw, so):
        wid = lax.axis_index("s") * NC + lax.axis_index("c")
        base = wid * RW
        pos_base = base % N
        pltpu.sync_copy(ids_hbm.at[pl.ds(base, RW)], idx_v)
        pltpu.sync_copy(g_hbm, gbuf)
        pltpu.sync_copy(b_hbm, bbuf)

        def slot_of(j):
            return lax.rem(j, NSLOT)

        def start_in(j):
            pltpu.async_copy(pt_hbm.at[pl.ds(pos_base + j * C, C)],
                             pbuf.at[slot_of(j)], sp.at[slot_of(j)])
            pltpu.async_copy(wt_hbm.at[idx_v.at[pl.ds(j * C, C)]],
                             wbuf.at[slot_of(j)], sw.at[slot_of(j)])

        def wait_in(j):
            pltpu.make_async_copy(pt_hbm.at[pl.ds(pos_base + j * C, C)],
                                  pbuf.at[slot_of(j)], sp.at[slot_of(j)]).wait()
            pltpu.make_async_copy(wt_hbm.at[idx_v.at[pl.ds(j * C, C)]],
                                  wbuf.at[slot_of(j)], sw.at[slot_of(j)]).wait()

        def start_out(j):
            pltpu.async_copy(wbuf.at[slot_of(j)],
                             out_hbm.at[pl.ds(base + j * C, C)],
                             so.at[slot_of(j)])

        def wait_out(j):
            pltpu.make_async_copy(wbuf.at[slot_of(j)],
                                  out_hbm.at[pl.ds(base + j * C, C)],
                                  so.at[slot_of(j)]).wait()

        lanes = lax.iota(jnp.int32, L)
        zero = jnp.zeros((L,), jnp.float32)

        def compute(slot):
            @plsc.parallel_loop(0, C, unroll=4)
            def row_body(r):
                sa = [zero, zero]
                qa = [zero, zero]
                for k in range(KV):
                    x = (wbuf[slot, r, pl.ds(k * L, L)]
                         + pbuf[slot, r, pl.ds(k * L, L)])
                    wbuf[slot, r, pl.ds(k * L, L)] = x
                    sa[k % 2] = sa[k % 2] + x
                    qa[k % 2] = qa[k % 2] + x * x
                s = sa[0] + sa[1]
                q = qa[0] + qa[1]
                for m in (1, 2, 4, 8):
                    red[r, 0] = s
                    red[r, 1] = q
                    s = s + plsc.load_gather(red.at[r, 0], [lanes ^ m])
                    q = q + plsc.load_gather(red.at[r, 1], [lanes ^ m])
                mean = s * (1.0 / DIM)
                rstd = _rsqrt(q * (1.0 / DIM) - mean * mean + EPS)
                for k in range(KV):
                    x = wbuf[slot, r, pl.ds(k * L, L)]
                    g = gbuf[pl.ds(k * L, L)]
                    bb = bbuf[pl.ds(k * L, L)]
                    wbuf[slot, r, pl.ds(k * L, L)] = \
                        (x - mean) * rstd * g + bb

        start_in(0)
        start_in(1)

        def chunk(j, carry):
            @pl.when(j >= 2)
            def _():
                wait_out(j - 2)

            @pl.when(j + 2 < NCHUNK)
            def _():
                start_in(j + 2)

            wait_in(j)
            compute(slot_of(j))
            start_out(j)
            return carry

        lax.fori_loop(0, NCHUNK, chunk, 0)
        wait_out(NCHUNK - 2)
        wait_out(NCHUNK - 1)

    return body


def _broadcast_pos(pos_table, b, n):
    def body(p_ref, o_ref):
        o_ref[0] = p_ref[...]

    return pl.pallas_call(
        body,
        grid=(b,),
        in_specs=[pl.BlockSpec((n, DIM), lambda i: (0, 0))],
        out_specs=pl.BlockSpec((1, n, DIM), lambda i: (i, 0, 0)),
        out_shape=jax.ShapeDtypeStruct((b, n, DIM), jnp.float32),
    )(pos_table)


@jax.jit
def kernel(input_ids, word_table, pos_table, ln_gamma, ln_beta):
    b, n = input_ids.shape
    T = b * n
    ids = input_ids.reshape(T).astype(jnp.int32)
    sc = _make_sc_kernel(T, n, 16)
    out1 = sc(ids, word_table, pos_table, ln_gamma, ln_beta)
    out2 = _broadcast_pos(pos_table, b, n)
    return out1.reshape(b, n, DIM), out2


# speedup vs baseline: 1.8874x; 1.8874x over previous
import functools

import jax
import jax.numpy as jnp
from jax import lax
from jax.experimental import pallas as pl
from jax.experimental.pallas import tpu as pltpu
from jax.experimental.pallas import tpu_sc as plsc

DIM = 768
EPS = 1e-9
L = 16
KV = DIM // L
NC, NS = 2, 16
NW = NC * NS
NSLOT = 4
G4 = 4


def _rsqrt(v):
    i = lax.bitcast_convert_type(v, jnp.int32)
    i = jnp.full((L,), 0x5F3759DF, jnp.int32) - lax.shift_right_logical(i, 1)
    y = lax.bitcast_convert_type(i, jnp.float32)
    half = v * 0.5
    for _ in range(3):
        y = y * (1.5 - half * y * y)
    return y


def _make_sc_kernel(T, N, C):
    RW = T // NW
    NCHUNK = RW // C
    mesh = plsc.VectorSubcoreMesh(core_axis_name="c", subcore_axis_name="s")

    @functools.partial(
        pl.kernel,
        out_type=jax.ShapeDtypeStruct((T, DIM), jnp.float32),
        mesh=mesh,
        compiler_params=pltpu.CompilerParams(needs_layout_passes=False),
        scratch_types=[
            pltpu.VMEM((RW,), jnp.int32),
            pltpu.VMEM((NSLOT, C, DIM), jnp.float32),
            pltpu.VMEM((NSLOT, C, DIM), jnp.float32),
            pltpu.VMEM((DIM,), jnp.float32),
            pltpu.VMEM((DIM,), jnp.float32),
            pltpu.VMEM((C, 2, L), jnp.float32),
            pltpu.SemaphoreType.DMA((NSLOT,)),
            pltpu.SemaphoreType.DMA((NSLOT,)),
            pltpu.SemaphoreType.DMA((NSLOT,)),
        ],
    )
    def body(ids_hbm, wt_hbm, pt_hbm, g_hbm, b_hbm, out_hbm,
             idx_v, wbuf, pbuf, gbuf, bbuf, red, sp, sw, so):
        wid = lax.axis_index("s") * NC + lax.axis_index("c")
        base = wid * RW
        pos_base = base % N
        pltpu.sync_copy(ids_hbm.at[pl.ds(base, RW)], idx_v)
        pltpu.sync_copy(g_hbm, gbuf)
        pltpu.sync_copy(b_hbm, bbuf)

        def slot_of(j):
            return lax.rem(j, NSLOT)

        def start_in(j):
            pltpu.async_copy(pt_hbm.at[pl.ds(pos_base + j * C, C)],
                             pbuf.at[slot_of(j)], sp.at[slot_of(j)])
            pltpu.async_copy(wt_hbm.at[idx_v.at[pl.ds(j * C, C)]],
                             wbuf.at[slot_of(j)], sw.at[slot_of(j)])

        def wait_in(j):
            pltpu.make_async_copy(pt_hbm.at[pl.ds(pos_base + j * C, C)],
                                  pbuf.at[slot_of(j)], sp.at[slot_of(j)]).wait()
            pltpu.make_async_copy(wt_hbm.at[idx_v.at[pl.ds(j * C, C)]],
                                  wbuf.at[slot_of(j)], sw.at[slot_of(j)]).wait()

        def start_out(j):
            pltpu.async_copy(wbuf.at[slot_of(j)],
                             out_hbm.at[pl.ds(base + j * C, C)],
                             so.at[slot_of(j)])

        def wait_out(j):
            pltpu.make_async_copy(wbuf.at[slot_of(j)],
                                  out_hbm.at[pl.ds(base + j * C, C)],
                                  so.at[slot_of(j)]).wait()

        lanes = lax.iota(jnp.int32, L)
        zero = jnp.zeros((L,), jnp.float32)

        def compute(slot):
            @plsc.parallel_loop(0, C, unroll=2)
            def row_body(r):
                sa = [zero, zero]
                qa = [zero, zero]
                for k in range(KV):
                    x = (wbuf[slot, r, pl.ds(k * L, L)]
                         + pbuf[slot, r, pl.ds(k * L, L)])
                    wbuf[slot, r, pl.ds(k * L, L)] = x
                    sa[k % 2] = sa[k % 2] + x
                    qa[k % 2] = qa[k % 2] + x * x
                s = sa[0] + sa[1]
                q = qa[0] + qa[1]
                for m in (1, 2, 4, 8):
                    red[r, 0] = s
                    red[r, 1] = q
                    s = s + plsc.load_gather(red.at[r, 0], [lanes ^ m])
                    q = q + plsc.load_gather(red.at[r, 1], [lanes ^ m])
                mean = s * (1.0 / DIM)
                rstd = _rsqrt(q * (1.0 / DIM) - mean * mean + EPS)
                for k in range(KV):
                    x = wbuf[slot, r, pl.ds(k * L, L)]
                    g = gbuf[pl.ds(k * L, L)]
                    bb = bbuf[pl.ds(k * L, L)]
                    wbuf[slot, r, pl.ds(k * L, L)] = \
                        (x - mean) * rstd * g + bb

        start_in(0)
        start_in(1)

        def chunk(j, carry):
            @pl.when(j >= 2)
            def _():
                wait_out(j - 2)

            @pl.when(j + 2 < NCHUNK)
            def _():
                start_in(j + 2)

            wait_in(j)
            compute(slot_of(j))
            start_out(j)
            return carry

        lax.fori_loop(0, NCHUNK, chunk, 0)
        wait_out(NCHUNK - 2)
        wait_out(NCHUNK - 1)

    return body


def _broadcast_pos(pos_table, b, n):
    def body(p_ref, o_ref):
        o_ref[0] = p_ref[...]

    return pl.pallas_call(
        body,
        grid=(b,),
        in_specs=[pl.BlockSpec((n, DIM), lambda i: (0, 0))],
        out_specs=pl.BlockSpec((1, n, DIM), lambda i: (i, 0, 0)),
        out_shape=jax.ShapeDtypeStruct((b, n, DIM), jnp.float32),
    )(pos_table)


@jax.jit
def kernel(input_ids, word_table, pos_table, ln_gamma, ln_beta):
    b, n = input_ids.shape
    T = b * n
    ids = input_ids.reshape(T).astype(jnp.int32)
    sc = _make_sc_kernel(T, n, 16)
    out1 = sc(ids, word_table, pos_table, ln_gamma, ln_beta)
    out2 = _broadcast_pos(pos_table, b, n)
    return out1.reshape(b, n, DIM), out2


# speedup vs baseline: 1.9766x; 1.0473x over previous
import functools

import jax
import jax.numpy as jnp
from jax import lax
from jax.experimental import pallas as pl
from jax.experimental.pallas import tpu as pltpu
from jax.experimental.pallas import tpu_sc as plsc

DIM = 768
EPS = 1e-9
L = 16
KV = DIM // L
NC, NS = 2, 16
NW = NC * NS
NSLOT = 4
G4 = 4


def _rsqrt(v):
    i = lax.bitcast_convert_type(v, jnp.int32)
    i = jnp.full((L,), 0x5F3759DF, jnp.int32) - lax.shift_right_logical(i, 1)
    y = lax.bitcast_convert_type(i, jnp.float32)
    half = v * 0.5
    for _ in range(3):
        y = y * (1.5 - half * y * y)
    return y


def _make_sc_kernel(T, N, C):
    RW = T // NW
    NCHUNK = RW // C
    mesh = plsc.VectorSubcoreMesh(core_axis_name="c", subcore_axis_name="s")

    @functools.partial(
        pl.kernel,
        out_type=(
            jax.ShapeDtypeStruct((T, DIM), jnp.float32),
            jax.ShapeDtypeStruct((T, DIM), jnp.float32),
        ),
        mesh=mesh,
        compiler_params=pltpu.CompilerParams(needs_layout_passes=False),
        scratch_types=[
            pltpu.VMEM((RW,), jnp.int32),
            pltpu.VMEM((NSLOT, C, DIM), jnp.float32),
            pltpu.VMEM((NSLOT, C, DIM), jnp.float32),
            pltpu.VMEM((DIM,), jnp.float32),
            pltpu.VMEM((DIM,), jnp.float32),
            pltpu.VMEM((C, 2, L), jnp.float32),
            pltpu.SemaphoreType.DMA((NSLOT,)),
            pltpu.SemaphoreType.DMA((NSLOT,)),
            pltpu.SemaphoreType.DMA((NSLOT,)),
        ],
    )
    def body(ids_hbm, wt_hbm, pt_hbm, g_hbm, b_hbm, out_hbm, out2_hbm,
             idx_v, wbuf, pbuf, gbuf, bbuf, red, sp, sw, so):
        wid = lax.axis_index("s") * NC + lax.axis_index("c")
        base = wid * RW
        pos_base = base % N
        pltpu.sync_copy(ids_hbm.at[pl.ds(base, RW)], idx_v)
        pltpu.sync_copy(g_hbm, gbuf)
        pltpu.sync_copy(b_hbm, bbuf)

        def slot_of(j):
            return lax.rem(j, NSLOT)

        def start_in(j):
            pltpu.async_copy(pt_hbm.at[pl.ds(pos_base + j * C, C)],
                             pbuf.at[slot_of(j)], sp.at[slot_of(j)])
            pltpu.async_copy(wt_hbm.at[idx_v.at[pl.ds(j * C, C)]],
                             wbuf.at[slot_of(j)], sw.at[slot_of(j)])

        def wait_in(j):
            pltpu.make_async_copy(pt_hbm.at[pl.ds(pos_base + j * C, C)],
                                  pbuf.at[slot_of(j)], sp.at[slot_of(j)]).wait()
            pltpu.make_async_copy(wt_hbm.at[idx_v.at[pl.ds(j * C, C)]],
                                  wbuf.at[slot_of(j)], sw.at[slot_of(j)]).wait()

        def start_out(j):
            pltpu.async_copy(wbuf.at[slot_of(j)],
                             out_hbm.at[pl.ds(base + j * C, C)],
                             so.at[slot_of(j)])
            pltpu.async_copy(pbuf.at[slot_of(j)],
                             out2_hbm.at[pl.ds(base + j * C, C)],
                             so.at[slot_of(j)])

        def wait_out(j):
            pltpu.make_async_copy(wbuf.at[slot_of(j)],
                                  out_hbm.at[pl.ds(base + j * C, C)],
                                  so.at[slot_of(j)]).wait()
            pltpu.make_async_copy(pbuf.at[slot_of(j)],
                                  out2_hbm.at[pl.ds(base + j * C, C)],
                                  so.at[slot_of(j)]).wait()

        lanes = lax.iota(jnp.int32, L)
        zero = jnp.zeros((L,), jnp.float32)

        def compute(slot):
            @plsc.parallel_loop(0, C, unroll=2)
            def row_body(r):
                sa = [zero, zero]
                qa = [zero, zero]
                for k in range(KV):
                    x = (wbuf[slot, r, pl.ds(k * L, L)]
                         + pbuf[slot, r, pl.ds(k * L, L)])
                    wbuf[slot, r, pl.ds(k * L, L)] = x
                    sa[k % 2] = sa[k % 2] + x
                    qa[k % 2] = qa[k % 2] + x * x
                s = sa[0] + sa[1]
                q = qa[0] + qa[1]
                for m in (1, 2, 4, 8):
                    red[r, 0] = s
                    red[r, 1] = q
                    s = s + plsc.load_gather(red.at[r, 0], [lanes ^ m])
                    q = q + plsc.load_gather(red.at[r, 1], [lanes ^ m])
                mean = s * (1.0 / DIM)
                rstd = _rsqrt(q * (1.0 / DIM) - mean * mean + EPS)
                for k in range(KV):
                    x = wbuf[slot, r, pl.ds(k * L, L)]
                    g = gbuf[pl.ds(k * L, L)]
                    bb = bbuf[pl.ds(k * L, L)]
                    wbuf[slot, r, pl.ds(k * L, L)] = \
                        (x - mean) * rstd * g + bb

        start_in(0)
        start_in(1)

        def chunk(j, carry):
            @pl.when(j >= 2)
            def _():
                wait_out(j - 2)

            @pl.when(j + 2 < NCHUNK)
            def _():
                start_in(j + 2)

            wait_in(j)
            compute(slot_of(j))
            start_out(j)
            return carry

        lax.fori_loop(0, NCHUNK, chunk, 0)
        wait_out(NCHUNK - 2)
        wait_out(NCHUNK - 1)

    return body


@jax.jit
def kernel(input_ids, word_table, pos_table, ln_gamma, ln_beta):
    b, n = input_ids.shape
    T = b * n
    ids = input_ids.reshape(T).astype(jnp.int32)
    sc = _make_sc_kernel(T, n, 16)
    out1, out2 = sc(ids, word_table, pos_table, ln_gamma, ln_beta)
    return out1.reshape(b, n, DIM), out2.reshape(b, n, DIM)
